# baseline (speedup 1.0000x reference)
import functools

import jax
import jax.numpy as jnp
from jax.experimental import pallas as pl
from jax.experimental.pallas import tpu as pltpu

_E = 16
_K = 8
_FT = 128


def _moe_body(x_ref, wr_ref, br_ref, wg_ref, wu_ref, wd_ref,
              wgs_ref, wus_ref, wds_ref, out_ref, comb_ref):
    f = pl.program_id(0)
    e = pl.program_id(1)
    x = x_ref[...]

    @pl.when((f == 0) & (e == 0))
    def _router_and_init():
        logits = jnp.dot(x, wr_ref[...], preferred_element_type=jnp.float32)
        logits = logits + br_ref[...]
        m = jnp.max(logits, axis=-1, keepdims=True)
        ex = jnp.exp(logits - m)
        probs = ex / jnp.sum(ex, axis=-1, keepdims=True)
        lane = jax.lax.broadcasted_iota(jnp.int32, probs.shape, 1)
        p = probs
        sel = jnp.zeros_like(probs)
        for _ in range(_K):
            mx = jnp.max(p, axis=-1, keepdims=True)
            cand = jnp.where(p == mx, lane, _E)
            first = jnp.min(cand, axis=-1, keepdims=True)
            onehot = lane == first
            sel = jnp.where(onehot, probs, sel)
            p = jnp.where(onehot, -jnp.inf, p)
        comb_ref[...] = sel / jnp.sum(sel, axis=-1, keepdims=True)
        out_ref[...] = jnp.zeros_like(out_ref)

    @pl.when(e < _E)
    def _routed_expert():
        g = jnp.dot(x, wg_ref[0], preferred_element_type=jnp.float32)
        u = jnp.dot(x, wu_ref[0], preferred_element_type=jnp.float32)
        h = g * jax.nn.sigmoid(g) * u
        y = jnp.dot(h, wd_ref[0], preferred_element_type=jnp.float32)
        comb = comb_ref[...]
        emask = jax.lax.broadcasted_iota(jnp.int32, comb.shape, 1) == e
        w = jnp.sum(jnp.where(emask, comb, 0.0), axis=-1, keepdims=True)
        out_ref[...] += w * y

    @pl.when(e == _E)
    def _shared_expert():
        g = jnp.dot(x, wgs_ref[...], preferred_element_type=jnp.float32)
        u = jnp.dot(x, wus_ref[...], preferred_element_type=jnp.float32)
        h = g * jax.nn.sigmoid(g) * u
        out_ref[...] += jnp.dot(h, wds_ref[...], preferred_element_type=jnp.float32)


@functools.partial(jax.jit, static_argnames=("interpret",))
def _moe(x, W_router, b_router, Wg, Wu, Wd, Wg_s, Wu_s, Wd_s, interpret=False):
    T, D = x.shape
    E = W_router.shape[1]
    F = Wg.shape[2]
    nf = F // _FT
    eclamp = E - 1
    grid = (nf, E + 1)
    return pl.pallas_call(
        _moe_body,
        grid=grid,
        in_specs=[
            pl.BlockSpec((T, D), lambda f, e: (0, 0)),
            pl.BlockSpec((D, E), lambda f, e: (0, 0)),
            pl.BlockSpec((1, E), lambda f, e: (0, 0)),
            pl.BlockSpec((1, D, _FT), lambda f, e: (jnp.minimum(e, eclamp), 0, f)),
            pl.BlockSpec((1, D, _FT), lambda f, e: (jnp.minimum(e, eclamp), 0, f)),
            pl.BlockSpec((1, _FT, D), lambda f, e: (jnp.minimum(e, eclamp), f, 0)),
            pl.BlockSpec((D, _FT), lambda f, e: (0, f)),
            pl.BlockSpec((D, _FT), lambda f, e: (0, f)),
            pl.BlockSpec((_FT, D), lambda f, e: (f, 0)),
        ],
        out_specs=pl.BlockSpec((T, D), lambda f, e: (0, 0)),
        out_shape=jax.ShapeDtypeStruct((T, D), x.dtype),
        scratch_shapes=[pltpu.VMEM((T, E), jnp.float32)],
        compiler_params=pltpu.CompilerParams(
            dimension_semantics=("arbitrary", "arbitrary")),
        interpret=interpret,
    )(x, W_router, b_router.reshape(1, E), Wg, Wu, Wd, Wg_s, Wu_s, Wd_s)


def kernel(x, W_router, b_router, Wg, Wu, Wd, Wg_s, Wu_s, Wd_s):
    return _moe(x, W_router, b_router, Wg, Wu, Wd, Wg_s, Wu_s, Wd_s)


# speedup vs baseline: 1.0086x; 1.0086x over previous
import functools

import jax
import jax.numpy as jnp
from jax.experimental import pallas as pl
from jax.experimental.pallas import tpu as pltpu

_E = 16
_K = 8
_FT = 128


def _moe_body(x_ref, wr_ref, br_ref, wg_ref, wu_ref, wd_ref,
              wgs_ref, wus_ref, wds_ref, out_ref, comb_ref, xbf_ref):
    f = pl.program_id(0)
    e = pl.program_id(1)

    @pl.when((f == 0) & (e == 0))
    def _router_and_init():
        x = x_ref[...]
        xbf_ref[...] = x.astype(jnp.bfloat16)
        logits = jnp.dot(x, wr_ref[...], preferred_element_type=jnp.float32)
        logits = logits + br_ref[...]
        m = jnp.max(logits, axis=-1, keepdims=True)
        ex = jnp.exp(logits - m)
        probs = ex / jnp.sum(ex, axis=-1, keepdims=True)
        lane = jax.lax.broadcasted_iota(jnp.int32, probs.shape, 1)
        p = probs
        sel = jnp.zeros_like(probs)
        for _ in range(_K):
            mx = jnp.max(p, axis=-1, keepdims=True)
            cand = jnp.where(p == mx, lane, _E)
            first = jnp.min(cand, axis=-1, keepdims=True)
            onehot = lane == first
            sel = jnp.where(onehot, probs, sel)
            p = jnp.where(onehot, -jnp.inf, p)
        comb_ref[...] = sel / jnp.sum(sel, axis=-1, keepdims=True)
        out_ref[...] = jnp.zeros_like(out_ref)

    xbf = xbf_ref[...]

    @pl.when(e < _E)
    def _routed_expert():
        g = jnp.dot(xbf, wg_ref[0].astype(jnp.bfloat16),
                    preferred_element_type=jnp.float32)
        u = jnp.dot(xbf, wu_ref[0].astype(jnp.bfloat16),
                    preferred_element_type=jnp.float32)
        h = g * jax.nn.sigmoid(g) * u
        comb = comb_ref[...]
        emask = jax.lax.broadcasted_iota(jnp.int32, comb.shape, 1) == e
        w = jnp.sum(jnp.where(emask, comb, 0.0), axis=-1, keepdims=True)
        hw = (w * h).astype(jnp.bfloat16)
        out_ref[...] += jnp.dot(hw, wd_ref[0].astype(jnp.bfloat16),
                                preferred_element_type=jnp.float32)

    @pl.when(e == _E)
    def _shared_expert():
        g = jnp.dot(xbf, wgs_ref[...].astype(jnp.bfloat16),
                    preferred_element_type=jnp.float32)
        u = jnp.dot(xbf, wus_ref[...].astype(jnp.bfloat16),
                    preferred_element_type=jnp.float32)
        h = (g * jax.nn.sigmoid(g) * u).astype(jnp.bfloat16)
        out_ref[...] += jnp.dot(h, wds_ref[...].astype(jnp.bfloat16),
                                preferred_element_type=jnp.float32)


@functools.partial(jax.jit, static_argnames=("interpret",))
def _moe(x, W_router, b_router, Wg, Wu, Wd, Wg_s, Wu_s, Wd_s, interpret=False):
    T, D = x.shape
    E = W_router.shape[1]
    F = Wg.shape[2]
    nf = F // _FT
    eclamp = E - 1
    grid = (nf, E + 1)
    return pl.pallas_call(
        _moe_body,
        grid=grid,
        in_specs=[
            pl.BlockSpec((T, D), lambda f, e: (0, 0)),
            pl.BlockSpec((D, E), lambda f, e: (0, 0)),
            pl.BlockSpec((1, E), lambda f, e: (0, 0)),
            pl.BlockSpec((1, D, _FT), lambda f, e: (jnp.minimum(e, eclamp), 0, f)),
            pl.BlockSpec((1, D, _FT), lambda f, e: (jnp.minimum(e, eclamp), 0, f)),
            pl.BlockSpec((1, _FT, D), lambda f, e: (jnp.minimum(e, eclamp), f, 0)),
            pl.BlockSpec((D, _FT), lambda f, e: (0, f)),
            pl.BlockSpec((D, _FT), lambda f, e: (0, f)),
            pl.BlockSpec((_FT, D), lambda f, e: (f, 0)),
        ],
        out_specs=pl.BlockSpec((T, D), lambda f, e: (0, 0)),
        out_shape=jax.ShapeDtypeStruct((T, D), x.dtype),
        scratch_shapes=[pltpu.VMEM((T, E), jnp.float32),
                        pltpu.VMEM((T, D), jnp.bfloat16)],
        compiler_params=pltpu.CompilerParams(
            dimension_semantics=("arbitrary", "arbitrary")),
        interpret=interpret,
    )(x, W_router, b_router.reshape(1, E), Wg, Wu, Wd, Wg_s, Wu_s, Wd_s)


def kernel(x, W_router, b_router, Wg, Wu, Wd, Wg_s, Wu_s, Wd_s):
    return _moe(x, W_router, b_router, Wg, Wu, Wd, Wg_s, Wu_s, Wd_s)
